# baseline (speedup 1.0000x reference)
import functools

import jax
import jax.numpy as jnp
from jax import lax
from jax.experimental import pallas as pl
from jax.experimental.pallas import tpu as pltpu
from jax.experimental.pallas import tpu_sc as plsc

_info = plsc.get_sparse_core_info()
_NC, _NS, _L = _info.num_cores, _info.num_subcores, _info.num_lanes
_NW = _NC * _NS

_B = 1024
_S = 20
_V = 64
_E = 64
_GPAD = 8
_TOK = _B * _S
_TPT = _TOK // _NW
_CHUNK = 128
_NCHUNK = _TPT // _CHUNK

_mesh = plsc.VectorSubcoreMesh(core_axis_name="c", subcore_axis_name="s")
_sc_params = pltpu.CompilerParams(use_tc_tiling_on_sc=False)


def _emb_body(blk_ids, lane_ids, *refs):
    tt_blks = refs[:_GPAD]
    cntb_blk, emb_blk, acc = refs[_GPAD], refs[_GPAD + 1], refs[_GPAD + 2]
    w = pl.program_id(0)
    iota = lax.broadcasted_iota(jnp.int32, (1, 128), 1)
    row = jnp.zeros((1, _E), jnp.float32)
    for k in range(_GPAD):
        lane = lane_ids[w * _GPAD + k]
        onehot = (iota == lane).astype(jnp.float32)
        row = row + lax.dot_general(onehot, tt_blks[k][...],
                                    (((1,), (1,)), ((), ())),
                                    preferred_element_type=jnp.float32)
    wmask = lax.broadcasted_iota(jnp.int32, (_V, _E), 0) == w
    upd = jnp.where(wmask, jnp.broadcast_to(row, (_V, _E)), 0.0)

    @pl.when(w == 0)
    def _():
        acc[...] = upd

    @pl.when(w > 0)
    def _():
        acc[...] = acc[...] + upd

    @pl.when(w == _V - 1)
    def _():
        emb_blk[...] = acc[...] / cntb_blk[...]


def _make_tt_spec(k):
    return pl.BlockSpec((_E, 128), lambda w, blk, lane, _k=k: (0, blk[w * _GPAD + _k]))


_build_emb = pl.pallas_call(
    _emb_body,
    grid_spec=pltpu.PrefetchScalarGridSpec(
        num_scalar_prefetch=2,
        grid=(_V,),
        in_specs=[_make_tt_spec(k) for k in range(_GPAD)]
        + [pl.BlockSpec((_V, _E), lambda w, blk, lane: (0, 0))],
        out_specs=pl.BlockSpec((_V, _E), lambda w, blk, lane: (0, 0)),
        scratch_shapes=[pltpu.VMEM((_V, _E), jnp.float32)],
    ),
    out_shape=jax.ShapeDtypeStruct((_V, _E), jnp.float32),
)


@functools.partial(
    pl.kernel,
    mesh=_mesh,
    compiler_params=_sc_params,
    out_type=jax.ShapeDtypeStruct((_TOK, _E), jnp.float32),
    scratch_types=[
        pltpu.VMEM((_NCHUNK, _CHUNK), jnp.int32),
        pltpu.VMEM((_TPT, _E), jnp.float32),
        pltpu.SemaphoreType.DMA,
    ],
)
def _expand(emb_hbm, idx_hbm, out_hbm, idx_v, rows_v, sem):
    wid = lax.axis_index("s") * _NC + lax.axis_index("c")
    pltpu.sync_copy(idx_hbm.at[wid], idx_v)
    copies = []
    for j in range(_NCHUNK):
        copies.append(
            pltpu.async_copy(
                emb_hbm.at[idx_v.at[j]],
                rows_v.at[pl.ds(j * _CHUNK, _CHUNK)],
                sem,
            )
        )
    for c in copies:
        c.wait()
    pltpu.sync_copy(rows_v, out_hbm.at[pl.ds(wid * _TPT, _TPT)])


def _fmt_body(src_blk, dst_blk):
    dst_blk[0] = jnp.transpose(src_blk[0], (1, 0))


_fmt = pl.pallas_call(
    _fmt_body,
    grid=(_S,),
    in_specs=[pl.BlockSpec((1, _B, _E), lambda s: (s, 0, 0))],
    out_specs=pl.BlockSpec((1, _E, _B), lambda s: (s, 0, 0)),
    out_shape=jax.ShapeDtypeStruct((_S, _E, _B), jnp.float32),
)


def kernel(word_idx, table, ngram_idx, ngram_cnt):
    tt = table.T
    idxp = jnp.pad(ngram_idx, ((0, 0), (0, _GPAD - ngram_idx.shape[1])))
    idxf = idxp.reshape(_V * _GPAD)
    blk_ids = idxf // 128
    lane_ids = idxf % 128
    cntb = jnp.broadcast_to(ngram_cnt[:, None], (_V, _E))
    emb = _build_emb(blk_ids, lane_ids, *([tt] * _GPAD), cntb)
    tok_idx = word_idx.T.reshape(_NW, _NCHUNK, _CHUNK)
    out_sb = _expand(emb, tok_idx)
    out3 = _fmt(out_sb.reshape(_S, _B, _E))
    return out3.transpose(2, 0, 1)


# speedup vs baseline: 1.0023x; 1.0023x over previous
import functools

import jax
import jax.numpy as jnp
from jax import lax
from jax.experimental import pallas as pl
from jax.experimental.pallas import tpu as pltpu
from jax.experimental.pallas import tpu_sc as plsc

_info = plsc.get_sparse_core_info()
_NC, _NS, _L = _info.num_cores, _info.num_subcores, _info.num_lanes
_NW = _NC * _NS

_B = 1024
_S = 20
_V = 64
_E = 64
_GPAD = 8
_GREAL = 6
_TOK = _B * _S
_TPT = _TOK // _NW
_CHUNK = 128
_NCHUNK = _TPT // _CHUNK

_mesh = plsc.VectorSubcoreMesh(core_axis_name="c", subcore_axis_name="s")
_sc_params = pltpu.CompilerParams(use_tc_tiling_on_sc=False)


def _emb_body(blk_ids, lane_ids, *refs):
    tt_blks = refs[:_GREAL]
    cntb_blk, emb_blk, acc = refs[_GREAL], refs[_GREAL + 1], refs[_GREAL + 2]
    w = pl.program_id(0)
    iota = lax.broadcasted_iota(jnp.int32, (1, 128), 1)
    row = jnp.zeros((1, _E), jnp.float32)
    for k in range(_GREAL):
        lane = lane_ids[w * _GPAD + k]
        onehot = (iota == lane).astype(jnp.float32)
        row = row + lax.dot_general(onehot, tt_blks[k][...],
                                    (((1,), (1,)), ((), ())),
                                    preferred_element_type=jnp.float32)
    wmask = lax.broadcasted_iota(jnp.int32, (_V, _E), 0) == w
    upd = jnp.where(wmask, jnp.broadcast_to(row, (_V, _E)), 0.0)

    @pl.when(w == 0)
    def _():
        acc[...] = upd

    @pl.when(w > 0)
    def _():
        acc[...] = acc[...] + upd

    @pl.when(w == _V - 1)
    def _():
        emb_blk[...] = acc[...] / cntb_blk[...]


def _make_tt_spec(k):
    return pl.BlockSpec((_E, 128), lambda w, blk, lane, _k=k: (0, blk[w * _GPAD + _k]))


_build_emb = pl.pallas_call(
    _emb_body,
    grid_spec=pltpu.PrefetchScalarGridSpec(
        num_scalar_prefetch=2,
        grid=(_V,),
        in_specs=[_make_tt_spec(k) for k in range(_GREAL)]
        + [pl.BlockSpec((_V, _E), lambda w, blk, lane: (0, 0))],
        out_specs=pl.BlockSpec((_V, _E), lambda w, blk, lane: (0, 0)),
        scratch_shapes=[pltpu.VMEM((_V, _E), jnp.float32)],
    ),
    out_shape=jax.ShapeDtypeStruct((_V, _E), jnp.float32),
)


@functools.partial(
    pl.kernel,
    mesh=_mesh,
    compiler_params=_sc_params,
    out_type=jax.ShapeDtypeStruct((_TOK, _E), jnp.float32),
    scratch_types=[
        pltpu.VMEM((_NCHUNK, _CHUNK), jnp.int32),
        pltpu.VMEM((_TPT, _E), jnp.float32),
        pltpu.SemaphoreType.DMA,
    ],
)
def _expand(emb_hbm, idx_hbm, out_hbm, idx_v, rows_v, sem):
    wid = lax.axis_index("s") * _NC + lax.axis_index("c")
    pltpu.sync_copy(idx_hbm.at[wid], idx_v)
    copies = []
    for j in range(_NCHUNK):
        copies.append(
            pltpu.async_copy(
                emb_hbm.at[idx_v.at[j]],
                rows_v.at[pl.ds(j * _CHUNK, _CHUNK)],
                sem,
            )
        )
    for c in copies:
        c.wait()
    pltpu.sync_copy(rows_v, out_hbm.at[pl.ds(wid * _TPT, _TPT)])


def _fmt_body(src_blk, dst_blk):
    dst_blk[0] = jnp.transpose(src_blk[0], (1, 0))


_fmt = pl.pallas_call(
    _fmt_body,
    grid=(_S,),
    in_specs=[pl.BlockSpec((1, _B, _E), lambda s: (s, 0, 0))],
    out_specs=pl.BlockSpec((1, _E, _B), lambda s: (s, 0, 0)),
    out_shape=jax.ShapeDtypeStruct((_S, _E, _B), jnp.float32),
)


def kernel(word_idx, table, ngram_idx, ngram_cnt):
    tt = table.T
    idxp = jnp.pad(ngram_idx, ((0, 0), (0, _GPAD - ngram_idx.shape[1])))
    idxf = idxp.reshape(_V * _GPAD)
    blk_ids = idxf // 128
    lane_ids = idxf % 128
    cntb = jnp.broadcast_to(ngram_cnt[:, None], (_V, _E))
    emb = _build_emb(blk_ids, lane_ids, *([tt] * _GREAL), cntb)
    tok_idx = word_idx.T.reshape(_NW, _NCHUNK, _CHUNK)
    out_sb = _expand(emb, tok_idx)
    out3 = _fmt(out_sb.reshape(_S, _B, _E))
    return out3.transpose(2, 0, 1)


# speedup vs baseline: 1.1084x; 1.1058x over previous
import functools

import jax
import jax.numpy as jnp
from jax import lax
from jax.experimental import pallas as pl
from jax.experimental.pallas import tpu as pltpu
from jax.experimental.pallas import tpu_sc as plsc

_info = plsc.get_sparse_core_info()
_NC, _NS, _L = _info.num_cores, _info.num_subcores, _info.num_lanes
_NW = _NC * _NS

_B = 1024
_S = 20
_V = 64
_E = 64
_GPAD = 8
_GREAL = 6
_TOK = _B * _S
_TPT = _TOK // _NW
_CHUNK = 128
_NCHUNK = _TPT // _CHUNK

_mesh = plsc.VectorSubcoreMesh(core_axis_name="c", subcore_axis_name="s")
_sc_params = pltpu.CompilerParams(use_tc_tiling_on_sc=False)


def _emb_body(blk_ids, lane_ids, *refs):
    tt_blks = refs[:_GREAL]
    cntb_blk, emb_blk, acc = refs[_GREAL], refs[_GREAL + 1], refs[_GREAL + 2]
    w = pl.program_id(0)
    iota = lax.broadcasted_iota(jnp.int32, (1, 128), 1)
    row = jnp.zeros((1, _E), jnp.float32)
    for k in range(_GREAL):
        lane = lane_ids[w * _GPAD + k]
        onehot = (iota == lane).astype(jnp.float32)
        row = row + lax.dot_general(onehot, tt_blks[k][...],
                                    (((1,), (1,)), ((), ())),
                                    preferred_element_type=jnp.float32)
    wmask = lax.broadcasted_iota(jnp.int32, (_V, _E), 0) == w
    upd = jnp.where(wmask, jnp.broadcast_to(row, (_V, _E)), 0.0)

    @pl.when(w == 0)
    def _():
        acc[...] = upd

    @pl.when(w > 0)
    def _():
        acc[...] = acc[...] + upd

    @pl.when(w == _V - 1)
    def _():
        emb_blk[...] = acc[...] / cntb_blk[...]


def _make_tt_spec(k):
    return pl.BlockSpec((_E, 128), lambda w, blk, lane, _k=k: (0, blk[w * _GPAD + _k]))


_build_emb = pl.pallas_call(
    _emb_body,
    grid_spec=pltpu.PrefetchScalarGridSpec(
        num_scalar_prefetch=2,
        grid=(_V,),
        in_specs=[_make_tt_spec(k) for k in range(_GREAL)]
        + [pl.BlockSpec((_V, _E), lambda w, blk, lane: (0, 0))],
        out_specs=pl.BlockSpec((_V, _E), lambda w, blk, lane: (0, 0)),
        scratch_shapes=[pltpu.VMEM((_V, _E), jnp.float32)],
    ),
    out_shape=jax.ShapeDtypeStruct((_V, _E), jnp.float32),
)


@functools.partial(
    pl.kernel,
    mesh=_mesh,
    compiler_params=_sc_params,
    out_type=jax.ShapeDtypeStruct((_TOK, _E), jnp.float32),
    scratch_types=[
        pltpu.VMEM((_NCHUNK, _CHUNK), jnp.int32),
        pltpu.VMEM((_TPT, _E), jnp.float32),
        pltpu.SemaphoreType.DMA,
    ],
)
def _expand(emb_hbm, idx_hbm, out_hbm, idx_v, rows_v, sem):
    wid = lax.axis_index("s") * _NC + lax.axis_index("c")
    pltpu.sync_copy(idx_hbm.at[wid], idx_v)
    copies = []
    for j in range(_NCHUNK):
        copies.append(
            pltpu.async_copy(
                emb_hbm.at[idx_v.at[j]],
                rows_v.at[pl.ds(j * _CHUNK, _CHUNK)],
                sem,
            )
        )
    for c in copies:
        c.wait()
    pltpu.sync_copy(rows_v, out_hbm.at[pl.ds(wid * _TPT, _TPT)])


def _fmt_body(src_blk, dst_blk):
    dst_blk[0] = jnp.transpose(src_blk[0], (1, 0))


_fmt = pl.pallas_call(
    _fmt_body,
    grid=(_S,),
    in_specs=[pl.BlockSpec((1, _B, _E), lambda s: (s, 0, 0))],
    out_specs=pl.BlockSpec((1, _E, _B), lambda s: (s, 0, 0)),
    out_shape=jax.ShapeDtypeStruct((_S, _E, _B), jnp.float32),
)


def kernel(word_idx, table, ngram_idx, ngram_cnt):
    tt = table.T
    idxp = jnp.pad(ngram_idx, ((0, 0), (0, _GPAD - ngram_idx.shape[1])))
    idxf = idxp.reshape(_V * _GPAD)
    blk_ids = idxf // 128
    lane_ids = idxf % 128
    cntb = jnp.broadcast_to(ngram_cnt[:, None], (_V, _E))
    emb = _build_emb(blk_ids, lane_ids, *([tt] * _GREAL), cntb)
    tok_idx = word_idx.reshape(_NW, _NCHUNK, _CHUNK)
    out = _expand(emb, tok_idx)
    return out.reshape(word_idx.shape + (_E,))


# speedup vs baseline: 1.2634x; 1.1398x over previous
import functools

import jax
import jax.numpy as jnp
from jax import lax
from jax.experimental import pallas as pl
from jax.experimental.pallas import tpu as pltpu
from jax.experimental.pallas import tpu_sc as plsc

_info = plsc.get_sparse_core_info()
_NC, _NS, _L = _info.num_cores, _info.num_subcores, _info.num_lanes
_NW = _NC * _NS

_B = 1024
_S = 20
_V = 64
_E = 64
_GPAD = 8
_GREAL = 6
_TOK = _B * _S
_TPT = _TOK // _NW
_CHUNK = 128
_NCHUNK = _TPT // _CHUNK

_mesh = plsc.VectorSubcoreMesh(core_axis_name="c", subcore_axis_name="s")
_sc_params = pltpu.CompilerParams(use_tc_tiling_on_sc=False)


_WPG = 4


def _emb_body(blk_ids, lane_ids, *refs):
    nblk = _WPG * _GREAL
    tt_blks = refs[:nblk]
    cntb_blk, emb_blk, acc = refs[nblk], refs[nblk + 1], refs[nblk + 2]
    g = pl.program_id(0)
    iota = lax.broadcasted_iota(jnp.int32, (1, 128), 1)
    upd = jnp.zeros((_V, _E), jnp.float32)
    for j in range(_WPG):
        w = g * _WPG + j
        row = jnp.zeros((1, _E), jnp.float32)
        for k in range(_GREAL):
            lane = lane_ids[w * _GPAD + k]
            onehot = (iota == lane).astype(jnp.float32)
            row = row + lax.dot_general(onehot, tt_blks[j * _GREAL + k][...],
                                        (((1,), (1,)), ((), ())),
                                        preferred_element_type=jnp.float32)
        wmask = lax.broadcasted_iota(jnp.int32, (_V, _E), 0) == w
        upd = upd + jnp.where(wmask, jnp.broadcast_to(row, (_V, _E)), 0.0)

    @pl.when(g == 0)
    def _():
        acc[...] = upd

    @pl.when(g > 0)
    def _():
        acc[...] = acc[...] + upd

    @pl.when(g == _V // _WPG - 1)
    def _():
        emb_blk[...] = acc[...] / cntb_blk[...]


def _make_tt_spec(j, k):
    def im(g, blk, lane, _j=j, _k=k):
        return (0, blk[(g * _WPG + _j) * _GPAD + _k])
    return pl.BlockSpec((_E, 128), im)


_build_emb = pl.pallas_call(
    _emb_body,
    grid_spec=pltpu.PrefetchScalarGridSpec(
        num_scalar_prefetch=2,
        grid=(_V // _WPG,),
        in_specs=[_make_tt_spec(j, k) for j in range(_WPG) for k in range(_GREAL)]
        + [pl.BlockSpec((_V, _E), lambda g, blk, lane: (0, 0))],
        out_specs=pl.BlockSpec((_V, _E), lambda g, blk, lane: (0, 0)),
        scratch_shapes=[pltpu.VMEM((_V, _E), jnp.float32)],
    ),
    out_shape=jax.ShapeDtypeStruct((_V, _E), jnp.float32),
)


@functools.partial(
    pl.kernel,
    mesh=_mesh,
    compiler_params=_sc_params,
    out_type=jax.ShapeDtypeStruct((_TOK, _E), jnp.float32),
    scratch_types=[
        pltpu.VMEM((_NCHUNK, _CHUNK), jnp.int32),
        pltpu.VMEM((_TPT, _E), jnp.float32),
        pltpu.SemaphoreType.DMA,
    ],
)
def _expand(emb_hbm, idx_hbm, out_hbm, idx_v, rows_v, sem):
    wid = lax.axis_index("s") * _NC + lax.axis_index("c")
    pltpu.sync_copy(idx_hbm.at[wid], idx_v)
    copies = []
    for j in range(_NCHUNK):
        copies.append(
            pltpu.async_copy(
                emb_hbm.at[idx_v.at[j]],
                rows_v.at[pl.ds(j * _CHUNK, _CHUNK)],
                sem,
            )
        )
    for c in copies:
        c.wait()
    pltpu.sync_copy(rows_v, out_hbm.at[pl.ds(wid * _TPT, _TPT)])


def _fmt_body(src_blk, dst_blk):
    dst_blk[0] = jnp.transpose(src_blk[0], (1, 0))


_fmt = pl.pallas_call(
    _fmt_body,
    grid=(_S,),
    in_specs=[pl.BlockSpec((1, _B, _E), lambda s: (s, 0, 0))],
    out_specs=pl.BlockSpec((1, _E, _B), lambda s: (s, 0, 0)),
    out_shape=jax.ShapeDtypeStruct((_S, _E, _B), jnp.float32),
)


def kernel(word_idx, table, ngram_idx, ngram_cnt):
    tt = table.T
    idxp = jnp.pad(ngram_idx, ((0, 0), (0, _GPAD - ngram_idx.shape[1])))
    idxf = idxp.reshape(_V * _GPAD)
    blk_ids = idxf // 128
    lane_ids = idxf % 128
    cntb = jnp.broadcast_to(ngram_cnt[:, None], (_V, _E))
    emb = _build_emb(blk_ids, lane_ids, *([tt] * (_WPG * _GREAL)), cntb)
    tok_idx = word_idx.reshape(_NW, _NCHUNK, _CHUNK)
    out = _expand(emb, tok_idx)
    return out.reshape(word_idx.shape + (_E,))


# speedup vs baseline: 1.2841x; 1.0164x over previous
import functools

import jax
import jax.numpy as jnp
from jax import lax
from jax.experimental import pallas as pl
from jax.experimental.pallas import tpu as pltpu
from jax.experimental.pallas import tpu_sc as plsc

_info = plsc.get_sparse_core_info()
_NC, _NS, _L = _info.num_cores, _info.num_subcores, _info.num_lanes
_NW = _NC * _NS

_B = 1024
_S = 20
_V = 64
_E = 64
_GPAD = 8
_GREAL = 6
_TOK = _B * _S
_TPT = _TOK // _NW
_CHUNK = 128
_NCHUNK = _TPT // _CHUNK

_mesh = plsc.VectorSubcoreMesh(core_axis_name="c", subcore_axis_name="s")
_sc_params = pltpu.CompilerParams(use_tc_tiling_on_sc=False)


_WPG = 8


def _emb_body(blk_ids, lane_ids, *refs):
    nblk = _WPG * _GREAL
    tt_blks = refs[:nblk]
    cntb_blk, emb_blk, acc = refs[nblk], refs[nblk + 1], refs[nblk + 2]
    g = pl.program_id(0)
    iota = lax.broadcasted_iota(jnp.int32, (1, 128), 1)
    upd = jnp.zeros((_V, _E), jnp.float32)
    for j in range(_WPG):
        w = g * _WPG + j
        row = jnp.zeros((1, _E), jnp.float32)
        for k in range(_GREAL):
            lane = lane_ids[w * _GPAD + k]
            onehot = (iota == lane).astype(jnp.float32)
            row = row + lax.dot_general(onehot, tt_blks[j * _GREAL + k][...],
                                        (((1,), (1,)), ((), ())),
                                        preferred_element_type=jnp.float32)
        wmask = lax.broadcasted_iota(jnp.int32, (_V, _E), 0) == w
        upd = upd + jnp.where(wmask, jnp.broadcast_to(row, (_V, _E)), 0.0)

    @pl.when(g == 0)
    def _():
        acc[...] = upd

    @pl.when(g > 0)
    def _():
        acc[...] = acc[...] + upd

    @pl.when(g == _V // _WPG - 1)
    def _():
        emb_blk[...] = acc[...] / cntb_blk[...]


def _make_tt_spec(j, k):
    def im(g, blk, lane, _j=j, _k=k):
        return (0, blk[(g * _WPG + _j) * _GPAD + _k])
    return pl.BlockSpec((_E, 128), im)


_build_emb = pl.pallas_call(
    _emb_body,
    grid_spec=pltpu.PrefetchScalarGridSpec(
        num_scalar_prefetch=2,
        grid=(_V // _WPG,),
        in_specs=[_make_tt_spec(j, k) for j in range(_WPG) for k in range(_GREAL)]
        + [pl.BlockSpec((_V, _E), lambda g, blk, lane: (0, 0))],
        out_specs=pl.BlockSpec((_V, _E), lambda g, blk, lane: (0, 0)),
        scratch_shapes=[pltpu.VMEM((_V, _E), jnp.float32)],
    ),
    out_shape=jax.ShapeDtypeStruct((_V, _E), jnp.float32),
)


@functools.partial(
    pl.kernel,
    mesh=_mesh,
    compiler_params=_sc_params,
    out_type=jax.ShapeDtypeStruct((_TOK, _E), jnp.float32),
    scratch_types=[
        pltpu.VMEM((_NCHUNK, _CHUNK), jnp.int32),
        pltpu.VMEM((_TPT, _E), jnp.float32),
        pltpu.SemaphoreType.DMA,
    ],
)
def _expand(emb_hbm, idx_hbm, out_hbm, idx_v, rows_v, sem):
    wid = lax.axis_index("s") * _NC + lax.axis_index("c")
    pltpu.sync_copy(idx_hbm.at[wid], idx_v)
    copies = []
    for j in range(_NCHUNK):
        copies.append(
            pltpu.async_copy(
                emb_hbm.at[idx_v.at[j]],
                rows_v.at[pl.ds(j * _CHUNK, _CHUNK)],
                sem,
            )
        )
    for c in copies:
        c.wait()
    pltpu.sync_copy(rows_v, out_hbm.at[pl.ds(wid * _TPT, _TPT)])


def _fmt_body(src_blk, dst_blk):
    dst_blk[0] = jnp.transpose(src_blk[0], (1, 0))


_fmt = pl.pallas_call(
    _fmt_body,
    grid=(_S,),
    in_specs=[pl.BlockSpec((1, _B, _E), lambda s: (s, 0, 0))],
    out_specs=pl.BlockSpec((1, _E, _B), lambda s: (s, 0, 0)),
    out_shape=jax.ShapeDtypeStruct((_S, _E, _B), jnp.float32),
)


def kernel(word_idx, table, ngram_idx, ngram_cnt):
    tt = table.T
    idxp = jnp.pad(ngram_idx, ((0, 0), (0, _GPAD - ngram_idx.shape[1])))
    idxf = idxp.reshape(_V * _GPAD)
    blk_ids = idxf // 128
    lane_ids = idxf % 128
    cntb = jnp.broadcast_to(ngram_cnt[:, None], (_V, _E))
    emb = _build_emb(blk_ids, lane_ids, *([tt] * (_WPG * _GREAL)), cntb)
    tok_idx = word_idx.reshape(_NW, _NCHUNK, _CHUNK)
    out = _expand(emb, tok_idx)
    return out.reshape(word_idx.shape + (_E,))
